# Initial kernel scaffold; baseline (speedup 1.0000x reference)
import jax
import jax.numpy as jnp
from jax.experimental import pallas as pl


def kernel(r, m_idx, p_idx, node_paths, node_signs, eta_bg, eta_meta, eta_pers):
    raise NotImplementedError("write your pallas kernel here")


# speedup vs baseline: 12.4539x; 12.4539x over previous
import functools

import jax
import jax.numpy as jnp
from jax import lax
from jax.experimental import pallas as pl
from jax.experimental.pallas import tpu as pltpu
from jax.experimental.pallas import tpu_sc as plsc

B = 16384
L = 14
LP = 16
NP1 = 16384
PAD = NP1 - 1
NW = 32
ROWS = B // NW
EPW = ROWS * LP
CHUNK = 128
NCH = EPW // CHUNK

_f32 = jnp.float32
_i32 = jnp.int32


def _sc_gather_body(paths_hbm, midx_hbm, pidx_hbm, bg_hbm, meta_hbm, pers_hbm,
                    bg_out, meta_out, pers_out,
                    paths_v, m_v, p_v, bg_v, im_v, ip_v, vm_v, vp_v, bga_v,
                    sem):
    wid = lax.axis_index("s") * 2 + lax.axis_index("c")
    base = wid * ROWS

    pltpu.sync_copy(paths_hbm.at[pl.ds(base * LP, EPW)], paths_v)
    pltpu.sync_copy(midx_hbm.at[pl.ds(base, ROWS)], m_v)
    pltpu.sync_copy(pidx_hbm.at[pl.ds(base, ROWS)], p_v)
    pltpu.sync_copy(bg_hbm, bg_v)

    def build(j, carry):
        pv = paths_v[pl.ds(j * LP, LP)]
        jv = jnp.full((LP,), 0, _i32) + j
        mv = plsc.load_gather(m_v, [jv])
        qv = plsc.load_gather(p_v, [jv])
        im_v[pl.ds(j * LP, LP)] = mv * NP1 + pv
        ip_v[pl.ds(j * LP, LP)] = qv * NP1 + pv
        return carry

    lax.fori_loop(0, ROWS, build, 0)

    def fire(c, carry):
        pltpu.async_copy(meta_hbm.at[im_v.at[pl.ds(c * CHUNK, CHUNK)]],
                         vm_v.at[pl.ds(c * CHUNK, CHUNK)], sem)
        pltpu.async_copy(pers_hbm.at[ip_v.at[pl.ds(c * CHUNK, CHUNK)]],
                         vp_v.at[pl.ds(c * CHUNK, CHUNK)], sem)
        return carry

    lax.fori_loop(0, NCH, fire, 0)

    def bgather(j, carry):
        pv = paths_v[pl.ds(j * LP, LP)]
        bga_v[pl.ds(j * LP, LP)] = plsc.load_gather(bg_v, [pv])
        return carry

    lax.fori_loop(0, ROWS, bgather, 0)
    pltpu.sync_copy(bga_v, bg_out.at[pl.ds(base * LP, EPW)])

    pltpu.make_async_copy(meta_hbm.at[pl.ds(0, EPW)], vm_v, sem).wait()
    pltpu.make_async_copy(pers_hbm.at[pl.ds(0, EPW)], vp_v, sem).wait()
    pltpu.sync_copy(vm_v, meta_out.at[pl.ds(base * LP, EPW)])
    pltpu.sync_copy(vp_v, pers_out.at[pl.ds(base * LP, EPW)])


_sc_gather = pl.kernel(
    _sc_gather_body,
    out_type=[jax.ShapeDtypeStruct((B * LP,), _f32) for _ in range(3)],
    mesh=plsc.VectorSubcoreMesh(core_axis_name="c", subcore_axis_name="s"),
    compiler_params=pltpu.CompilerParams(needs_layout_passes=False),
    scratch_types=[
        pltpu.VMEM((EPW,), _i32),
        pltpu.VMEM((ROWS,), _i32),
        pltpu.VMEM((ROWS,), _i32),
        pltpu.VMEM((NP1,), _f32),
        pltpu.VMEM((EPW,), _i32),
        pltpu.VMEM((EPW,), _i32),
        pltpu.VMEM((EPW,), _f32),
        pltpu.VMEM((EPW,), _f32),
        pltpu.VMEM((EPW,), _f32),
        pltpu.SemaphoreType.DMA,
    ],
)

_TC_ROWS = B * LP // 128
_TC_BLK = 256


def _tc_finish_body(bg_ref, m_ref, p_ref, s_ref, q_ref, o_ref):
    x = s_ref[...] * (bg_ref[...] + m_ref[...] + p_ref[...])
    y = jnp.minimum(x, 0.0) - jnp.log(1.0 + jnp.exp(-jnp.abs(x)))
    z = y * (q_ref[...] != PAD).astype(_f32)
    row = lax.broadcasted_iota(_i32, (128, 8), 0) // LP
    col = lax.broadcasted_iota(_i32, (128, 8), 1)
    sel = (row == col).astype(_f32)
    o_ref[...] = jnp.dot(z, sel, preferred_element_type=_f32)


_tc_finish = pl.pallas_call(
    _tc_finish_body,
    grid=(_TC_ROWS // _TC_BLK,),
    in_specs=[pl.BlockSpec((_TC_BLK, 128), lambda i: (i, 0)) for _ in range(4)]
    + [pl.BlockSpec((_TC_BLK, 128), lambda i: (i, 0))],
    out_specs=pl.BlockSpec((_TC_BLK, 8), lambda i: (i, 0)),
    out_shape=jax.ShapeDtypeStruct((_TC_ROWS, 8), _f32),
)


def kernel(r, m_idx, p_idx, node_paths, node_signs, eta_bg, eta_meta, eta_pers):
    del r
    paths_p = jnp.pad(node_paths, ((0, 0), (0, LP - L)), constant_values=PAD)
    signs_p = jnp.pad(node_signs, ((0, 0), (0, LP - L)))
    bgv, mv, pv = _sc_gather(
        paths_p.reshape(-1),
        m_idx.astype(_i32),
        p_idx.astype(_i32),
        eta_bg,
        eta_meta.reshape(-1),
        eta_pers.reshape(-1),
    )
    out8 = _tc_finish(
        bgv.reshape(_TC_ROWS, 128),
        mv.reshape(_TC_ROWS, 128),
        pv.reshape(_TC_ROWS, 128),
        signs_p.reshape(_TC_ROWS, 128),
        paths_p.reshape(_TC_ROWS, 128),
    )
    return out8.reshape(B)
